# baseline (speedup 1.0000x reference)
import jax
import jax.numpy as jnp
from jax.experimental import pallas as pl
from jax.experimental.pallas import tpu as pltpu


def _dualhot_kernel(ids_a_ref, ids_b_ref, head_ref, table2_ref, out_ref):
    tp = head_ref.shape[0]
    R2 = table2_ref.shape[0]
    iota = jax.lax.broadcasted_iota(jnp.int32, (tp, R2), 1)
    dual_hot = ((iota == ids_a_ref[...]) | (iota == ids_b_ref[...])
                ).astype(jnp.bfloat16)
    gathered = jnp.dot(dual_hot, table2_ref[...],
                       preferred_element_type=jnp.float32)
    out_ref[...] = head_ref[...] + gathered


def kernel(head_embed, rel_ids, embed_table):
    B, D = head_embed.shape
    R, _ = embed_table.shape
    P = B // 2
    tp = 1024
    grid_b = pl.cdiv(P, tp)

    ids_1d = rel_ids.astype(jnp.int32)
    ids_a = ids_1d[0::2].reshape(P, 1)
    ids_b = (ids_1d[1::2] + R).reshape(P, 1)
    head_pairs = head_embed.reshape(P, 2 * D)
    tb16 = embed_table.astype(jnp.bfloat16)
    z = jnp.zeros((R, D), dtype=jnp.bfloat16)
    table2 = jnp.concatenate(
        [jnp.concatenate([tb16, z], axis=1),
         jnp.concatenate([z, tb16], axis=1)], axis=0)

    out = pl.pallas_call(
        _dualhot_kernel,
        out_shape=jax.ShapeDtypeStruct((P, 2 * D), head_embed.dtype),
        grid=(grid_b,),
        in_specs=[
            pl.BlockSpec((tp, 1), lambda i: (i, 0)),
            pl.BlockSpec((tp, 1), lambda i: (i, 0)),
            pl.BlockSpec((tp, 2 * D), lambda i: (i, 0)),
            pl.BlockSpec((2 * R, 2 * D), lambda i: (0, 0)),
        ],
        out_specs=pl.BlockSpec((tp, 2 * D), lambda i: (i, 0)),
        compiler_params=pltpu.CompilerParams(
            dimension_semantics=("parallel",),
        ),
    )(ids_a, ids_b, head_pairs, table2)
    return out.reshape(B, D)


# speedup vs baseline: 3.1438x; 3.1438x over previous
import jax
import jax.numpy as jnp
from jax.experimental import pallas as pl
from jax.experimental.pallas import tpu as pltpu

_CHUNKS = 16


def _onehot_kernel(ids_ref, head_ref, table_ref, out_ref):
    tb, D = head_ref.shape
    R = table_ref.shape[0]
    mc = tb // _CHUNKS
    iota_mc = jax.lax.broadcasted_iota(jnp.int32, (mc, R), 1)
    table = table_ref[...]
    dots = []
    for k in range(_CHUNKS):
        ids_k = ids_ref[pl.ds(k * mc, mc), :]
        one_hot = (iota_mc == ids_k).astype(jnp.bfloat16)
        dots.append(jnp.dot(one_hot, table, preferred_element_type=jnp.float32))
        if k >= 1:
            r0 = (k - 1) * mc
            out_ref[pl.ds(r0, mc), :] = head_ref[pl.ds(r0, mc), :] + dots[k - 1]
    r0 = (_CHUNKS - 1) * mc
    out_ref[pl.ds(r0, mc), :] = head_ref[pl.ds(r0, mc), :] + dots[-1]


def kernel(head_embed, rel_ids, embed_table):
    B, D = head_embed.shape
    R, _ = embed_table.shape
    tb = 4096
    grid_b = pl.cdiv(B, tb)
    ids_2d = rel_ids.astype(jnp.int32).reshape(B, 1)
    table_bf16 = embed_table.astype(jnp.bfloat16)

    return pl.pallas_call(
        _onehot_kernel,
        out_shape=jax.ShapeDtypeStruct((B, D), head_embed.dtype),
        grid=(grid_b,),
        in_specs=[
            pl.BlockSpec((tb, 1), lambda i: (i, 0)),
            pl.BlockSpec((tb, D), lambda i: (i, 0)),
            pl.BlockSpec((R, D), lambda i: (0, 0)),
        ],
        out_specs=pl.BlockSpec((tb, D), lambda i: (i, 0)),
        compiler_params=pltpu.CompilerParams(
            dimension_semantics=("parallel",),
        ),
    )(ids_2d, head_embed, table_bf16)


# speedup vs baseline: 3.2132x; 1.0221x over previous
import jax
import jax.numpy as jnp
from jax.experimental import pallas as pl
from jax.experimental.pallas import tpu as pltpu

_CHUNKS = 32


def _onehot_kernel(ids_ref, head_ref, table_ref, out_ref):
    tb, D = head_ref.shape
    R = table_ref.shape[0]
    mc = tb // _CHUNKS
    iota_mc = jax.lax.broadcasted_iota(jnp.int32, (mc, R), 1)
    table = table_ref[...]
    dots = []
    for k in range(_CHUNKS):
        ids_k = ids_ref[pl.ds(k * mc, mc), :]
        one_hot = (iota_mc == ids_k).astype(jnp.bfloat16)
        dots.append(jnp.dot(one_hot, table, preferred_element_type=jnp.float32))
        if k >= 1:
            r0 = (k - 1) * mc
            out_ref[pl.ds(r0, mc), :] = head_ref[pl.ds(r0, mc), :] + dots[k - 1]
    r0 = (_CHUNKS - 1) * mc
    out_ref[pl.ds(r0, mc), :] = head_ref[pl.ds(r0, mc), :] + dots[-1]


def kernel(head_embed, rel_ids, embed_table):
    B, D = head_embed.shape
    R, _ = embed_table.shape
    tb = 8192
    grid_b = pl.cdiv(B, tb)
    ids_2d = rel_ids.astype(jnp.int32).reshape(B, 1)
    table_bf16 = embed_table.astype(jnp.bfloat16)

    return pl.pallas_call(
        _onehot_kernel,
        out_shape=jax.ShapeDtypeStruct((B, D), head_embed.dtype),
        grid=(grid_b,),
        in_specs=[
            pl.BlockSpec((tb, 1), lambda i: (i, 0)),
            pl.BlockSpec((tb, D), lambda i: (i, 0)),
            pl.BlockSpec((R, D), lambda i: (0, 0)),
        ],
        out_specs=pl.BlockSpec((tb, D), lambda i: (i, 0)),
        compiler_params=pltpu.CompilerParams(
            dimension_semantics=("parallel",),
        ),
    )(ids_2d, head_embed, table_bf16)


# speedup vs baseline: 3.2151x; 1.0006x over previous
import jax
import jax.numpy as jnp
from jax.experimental import pallas as pl
from jax.experimental.pallas import tpu as pltpu

_CHUNKS = 16


def _onehot_kernel(ids_ref, head_ref, table_ref, out_ref):
    tb, D = head_ref.shape
    R = table_ref.shape[0]
    mc = tb // _CHUNKS
    iota_mc = jax.lax.broadcasted_iota(jnp.int32, (mc, R), 1)
    table = table_ref[...]
    dots = []
    for k in range(_CHUNKS):
        ids_k = ids_ref[pl.ds(k * mc, mc), :]
        one_hot = (iota_mc == ids_k).astype(jnp.bfloat16)
        dots.append(jnp.dot(one_hot, table, preferred_element_type=jnp.float32))
        if k >= 1:
            r0 = (k - 1) * mc
            out_ref[pl.ds(r0, mc), :] = head_ref[pl.ds(r0, mc), :] + dots[k - 1]
    r0 = (_CHUNKS - 1) * mc
    out_ref[pl.ds(r0, mc), :] = head_ref[pl.ds(r0, mc), :] + dots[-1]


def kernel(head_embed, rel_ids, embed_table):
    B, D = head_embed.shape
    R, _ = embed_table.shape
    tb = 8192
    grid_b = pl.cdiv(B, tb)
    ids_2d = rel_ids.astype(jnp.int32).reshape(B, 1)
    table_bf16 = embed_table.astype(jnp.bfloat16)

    return pl.pallas_call(
        _onehot_kernel,
        out_shape=jax.ShapeDtypeStruct((B, D), head_embed.dtype),
        grid=(grid_b,),
        in_specs=[
            pl.BlockSpec((tb, 1), lambda i: (i, 0)),
            pl.BlockSpec((tb, D), lambda i: (i, 0)),
            pl.BlockSpec((R, D), lambda i: (0, 0)),
        ],
        out_specs=pl.BlockSpec((tb, D), lambda i: (i, 0)),
        compiler_params=pltpu.CompilerParams(
            dimension_semantics=("parallel",),
        ),
    )(ids_2d, head_embed, table_bf16)


# speedup vs baseline: 3.2184x; 1.0010x over previous
import functools

import jax
import jax.numpy as jnp
from jax.experimental import pallas as pl
from jax.experimental.pallas import tpu as pltpu

_TILE_CANDIDATES = (8192, 4096, 2048, 1024, 512, 256, 128, 64, 32, 16, 8)


def _onehot_matmul_kernel(ids_ref, head_ref, table_ref, out_ref, *, chunks):
    tb, D = head_ref.shape
    R = table_ref.shape[0]
    mc = tb // chunks
    iota_mc = jax.lax.broadcasted_iota(jnp.int32, (mc, R), 1)
    table = table_ref[...]
    dots = []
    for k in range(chunks):
        ids_k = ids_ref[pl.ds(k * mc, mc), :]
        one_hot = (iota_mc == ids_k).astype(jnp.bfloat16)
        dots.append(jnp.dot(one_hot, table,
                            preferred_element_type=jnp.float32))
        if k >= 1:
            r0 = (k - 1) * mc
            out_ref[pl.ds(r0, mc), :] = head_ref[pl.ds(r0, mc), :] + dots[k - 1]
    r0 = (chunks - 1) * mc
    out_ref[pl.ds(r0, mc), :] = head_ref[pl.ds(r0, mc), :] + dots[-1]


def kernel(head_embed, rel_ids, embed_table):
    B, D = head_embed.shape
    R, _ = embed_table.shape
    tb = next((t for t in _TILE_CANDIDATES if B % t == 0), min(B, 8))
    chunks = max(1, min(16, tb // 8))
    grid_b = pl.cdiv(B, tb)

    ids_2d = rel_ids.astype(jnp.int32).reshape(B, 1)
    table_bf16 = embed_table.astype(jnp.bfloat16)
    body = functools.partial(_onehot_matmul_kernel, chunks=chunks)

    return pl.pallas_call(
        body,
        out_shape=jax.ShapeDtypeStruct((B, D), head_embed.dtype),
        grid=(grid_b,),
        in_specs=[
            pl.BlockSpec((tb, 1), lambda i: (i, 0)),
            pl.BlockSpec((tb, D), lambda i: (i, 0)),
            pl.BlockSpec((R, D), lambda i: (0, 0)),
        ],
        out_specs=pl.BlockSpec((tb, D), lambda i: (i, 0)),
        compiler_params=pltpu.CompilerParams(
            dimension_semantics=("parallel",),
        ),
    )(ids_2d, head_embed, table_bf16)
